# baseline (speedup 1.0000x reference)
import functools

import jax
import jax.numpy as jnp
from jax import lax
from jax.experimental import pallas as pl
from jax.experimental.pallas import tpu as pltpu
from jax.experimental.pallas import tpu_sc as plsc

H = 1024
FF = 4096
E = 8
TOPK = 2
TM = 256
T = 4096
MT = T // TM + E
APAD = MT * TM
RB = 128
FC = 1024


def _routing_body(x_ref, pos_ref, g_ref, oh_ref, rank_ref, h_ref):
    f32 = jnp.float32
    s = x_ref[:, 0:1] + x_ref[:, 1:2]
    r = lax.rem(s.astype(jnp.int32), E)
    hv = jnp.where(r < 0, r + E, r)
    h_ref[...] = hv.astype(f32)

    ii = lax.broadcasted_iota(jnp.int32, (RB, RB), 0)
    jj = lax.broadcasted_iota(jnp.int32, (RB, RB), 1)
    L = (ii > jj).astype(f32)
    lane = jj.astype(f32)

    def loop1(j, run):
        hj = h_ref[pl.ds(j * RB, RB), :]
        ohj = (hj == lane).astype(f32)
        oh_ref[pl.ds(j * RB, RB), :] = ohj
        rank_ref[pl.ds(j * RB, RB), :] = (
            jnp.dot(L, ohj, preferred_element_type=f32)
            + jnp.broadcast_to(run, (RB, RB)))
        return run + jnp.sum(ohj, axis=0, keepdims=True)

    run = lax.fori_loop(0, T // RB, loop1,
                        jnp.zeros((1, RB), f32))

    tiles = jnp.floor((run + (TM - 1)) * (1.0 / TM))
    UP = (ii < jj).astype(f32)
    pad_off = jnp.dot(tiles, UP, preferred_element_type=f32) * TM

    cum_incl = pad_off * (1.0 / TM) + tiles
    ONES = jnp.ones((RB, RB), f32)
    D = (ii == jj).astype(f32) * jnp.broadcast_to(cum_incl, (RB, RB))
    cum_sub = jnp.dot(D, ONES, preferred_element_type=f32)
    gcnt = jnp.dot(jnp.ones((1, RB), f32), (cum_sub <= lane).astype(f32),
                   preferred_element_type=f32)
    g_row = jnp.minimum(gcnt, float(E - 1))
    total_row = jnp.dot(tiles, ONES, preferred_element_type=f32)
    g_row = jnp.where(jj[:1, :].astype(f32) == float(MT), total_row, g_row)
    g_ref[...] = jnp.broadcast_to(g_row, (E, RB)).astype(jnp.int32)

    def loop2(j, carry):
        ohj = oh_ref[pl.ds(j * RB, RB), :]
        rankj = rank_ref[pl.ds(j * RB, RB), :]
        pj = jnp.sum(ohj * (rankj + jnp.broadcast_to(pad_off, (RB, RB))),
                     axis=1, keepdims=True)
        pos_ref[pl.ds(j * RB, RB), :] = pj.astype(jnp.int32)
        return carry

    lax.fori_loop(0, T // RB, loop2, 0)


def _routing_tc(x_flat):
    return pl.pallas_call(
        _routing_body,
        grid=(1,),
        in_specs=[pl.BlockSpec((T, RB), lambda i: (0, 0))],
        out_specs=[
            pl.BlockSpec((T, 1), lambda i: (0, 0)),
            pl.BlockSpec((E, RB), lambda i: (0, 0)),
        ],
        out_shape=[
            jax.ShapeDtypeStruct((T, 1), jnp.int32),
            jax.ShapeDtypeStruct((E, RB), jnp.int32),
        ],
        scratch_shapes=[
            pltpu.VMEM((T, RB), jnp.float32),
            pltpu.VMEM((T, RB), jnp.float32),
            pltpu.VMEM((T, 1), jnp.float32),
        ],
    )(x_flat[:, :RB])


def _sc_scatter_rows(rows, idx3, n_out):
    B, D = rows.shape
    info = plsc.get_sparse_core_info()
    nc, ns = info.num_cores, info.num_subcores
    nw = nc * ns
    b_per_w = B // nw
    nch = idx3.shape[1]
    ch = idx3.shape[2]
    mesh = plsc.VectorSubcoreMesh(core_axis_name="c", subcore_axis_name="s")

    @functools.partial(
        pl.kernel, mesh=mesh,
        out_type=jax.ShapeDtypeStruct((n_out, D), rows.dtype),
        scratch_types=[
            pltpu.VMEM((ch,), jnp.int32),
            pltpu.VMEM((ch, D), rows.dtype),
            pltpu.SemaphoreType.DMA,
        ],
    )
    def k(rows_hbm, idx_hbm, out_hbm, idx_v, rows_v, sem):
        wid = lax.axis_index("s") * nc + lax.axis_index("c")
        base = wid * b_per_w

        def body(j, carry):
            pltpu.sync_copy(idx_hbm.at[wid, j], idx_v)
            pltpu.sync_copy(rows_hbm.at[pl.ds(base + j * ch, ch)], rows_v)
            pltpu.async_copy(rows_v, out_hbm.at[idx_v], sem).wait()
            return carry

        lax.fori_loop(0, nch, body, 0)

    return k(rows, idx3)


def _sc_gather_rows(table, idx):
    B = idx.shape[0]
    D = table.shape[1]
    info = plsc.get_sparse_core_info()
    nc, ns = info.num_cores, info.num_subcores
    nw = nc * ns
    b_per_w = B // nw
    ch = 64 if b_per_w % 64 == 0 else b_per_w
    nch = b_per_w // ch
    mesh = plsc.VectorSubcoreMesh(core_axis_name="c", subcore_axis_name="s")

    @functools.partial(
        pl.kernel, mesh=mesh,
        out_type=jax.ShapeDtypeStruct((B, D), table.dtype),
        scratch_types=[
            pltpu.VMEM((ch,), jnp.int32),
            pltpu.VMEM((ch, D), table.dtype),
            pltpu.SemaphoreType.DMA,
        ],
    )
    def k(table_hbm, idx_hbm, out_hbm, idx_v, rows_v, sem):
        wid = lax.axis_index("s") * nc + lax.axis_index("c")
        base = wid * b_per_w

        def body(i, carry):
            off = base + i * ch
            pltpu.sync_copy(idx_hbm.at[pl.ds(off, ch)], idx_v)
            pltpu.async_copy(table_hbm.at[idx_v], rows_v, sem).wait()
            pltpu.sync_copy(rows_v, out_hbm.at[pl.ds(off, ch)])
            return carry

        lax.fori_loop(0, nch, body, 0)

    return k(table, idx)


def _mlp_pass_body(acc, add_bias):
    def body(g_ref, x_ref, w1_ref, b1_ref, w2_ref, b2_ref, *rest):
        if acc:
            y0_ref, o_ref = rest
        else:
            (o_ref,) = rest

        @pl.when(pl.program_id(0) < g_ref[MT])
        def _():
            xb = x_ref[...].astype(jnp.bfloat16)
            y = jnp.zeros((TM, H), jnp.float32)
            if add_bias:
                y = y + (0.5 * b2_ref[0, 0])[None, :]
            if acc:
                y = y + y0_ref[...]
            h = jnp.dot(xb, w1_ref[0], preferred_element_type=jnp.float32)
            h = (jax.nn.relu(h + b1_ref[0, 0][None, :]) * 0.5).astype(jnp.bfloat16)
            y = y + jnp.dot(h, w2_ref[0], preferred_element_type=jnp.float32)
            o_ref[...] = y
    return body


def _mlp_pass(g_tile, x_s, W1, b1r, W2, b2r, shift, y0=None):
    grid = (MT,)

    def e_idx(m, g):
        return lax.rem(g[m] + shift, E)

    in_specs = [
        pl.BlockSpec((TM, H), lambda m, g: (m, 0)),
        pl.BlockSpec((1, H, FF), lambda m, g: (e_idx(m, g), 0, 0)),
        pl.BlockSpec((1, 1, FF), lambda m, g: (e_idx(m, g), 0, 0)),
        pl.BlockSpec((1, FF, H), lambda m, g: (e_idx(m, g), 0, 0)),
        pl.BlockSpec((1, 1, H), lambda m, g: (e_idx(m, g), 0, 0)),
    ]
    args = [g_tile, x_s, W1, b1r, W2, b2r]
    if y0 is not None:
        in_specs.append(pl.BlockSpec((TM, H), lambda m, g: (m, 0)))
        args.append(y0)
    grid_spec = pltpu.PrefetchScalarGridSpec(
        num_scalar_prefetch=1,
        grid=grid,
        in_specs=in_specs,
        out_specs=pl.BlockSpec((TM, H), lambda m, g: (m, 0)),
    )
    return pl.pallas_call(
        _mlp_pass_body(y0 is not None, True),
        grid_spec=grid_spec,
        out_shape=jax.ShapeDtypeStruct((APAD, H), jnp.float32),
        compiler_params=pltpu.CompilerParams(
            dimension_semantics=("arbitrary",)),
    )(*args)


def kernel(x, W1, b1, W2, b2):
    B, S, _ = x.shape
    x_flat = x.reshape(-1, H)

    pos_col, g_out = _routing_tc(x_flat)
    pos = pos_col.reshape(T)
    g_tile = g_out[0, :MT + E]

    nw = 32
    idx3 = pos.reshape(nw, 2, T // nw // 2)
    x_s = _sc_scatter_rows(x_flat, idx3, APAD)

    w2b = W2.astype(jnp.bfloat16)
    b1r = b1.reshape(E, 1, FF)
    b2r = b2.reshape(E, 1, H)
    y0 = _mlp_pass(g_tile, x_s, W1, b1r, w2b, b2r, 0)
    y_s = _mlp_pass(g_tile, x_s, W1, b1r, w2b, b2r, 1, y0)

    out_flat = _sc_gather_rows(y_s, pos)
    return out_flat.reshape(B, S, H)


# speedup vs baseline: 1.0105x; 1.0105x over previous
import functools

import jax
import jax.numpy as jnp
from jax import lax
from jax.experimental import pallas as pl
from jax.experimental.pallas import tpu as pltpu
from jax.experimental.pallas import tpu_sc as plsc

H = 1024
FF = 4096
E = 8
TOPK = 2
TM = 256
T = 4096
MT = T // TM + E
APAD = MT * TM
RB = 128
FC = 1024


def _routing_body(x_ref, pos_ref, g_ref, oh_ref, rank_ref, h_ref):
    f32 = jnp.float32
    s = x_ref[:, 0:1] + x_ref[:, 1:2]
    r = lax.rem(s.astype(jnp.int32), E)
    hv = jnp.where(r < 0, r + E, r)
    h_ref[...] = hv.astype(f32)

    ii = lax.broadcasted_iota(jnp.int32, (RB, RB), 0)
    jj = lax.broadcasted_iota(jnp.int32, (RB, RB), 1)
    L = (ii > jj).astype(f32)
    lane = jj.astype(f32)

    def loop1(j, run):
        hj = h_ref[pl.ds(j * RB, RB), :]
        ohj = (hj == lane).astype(f32)
        oh_ref[pl.ds(j * RB, RB), :] = ohj
        rank_ref[pl.ds(j * RB, RB), :] = (
            jnp.dot(L, ohj, preferred_element_type=f32)
            + jnp.broadcast_to(run, (RB, RB)))
        return run + jnp.sum(ohj, axis=0, keepdims=True)

    run = lax.fori_loop(0, T // RB, loop1,
                        jnp.zeros((1, RB), f32))

    tiles = jnp.floor((run + (TM - 1)) * (1.0 / TM))
    UP = (ii < jj).astype(f32)
    pad_off = jnp.dot(tiles, UP, preferred_element_type=f32) * TM

    cum_incl = pad_off * (1.0 / TM) + tiles
    ONES = jnp.ones((RB, RB), f32)
    D = (ii == jj).astype(f32) * jnp.broadcast_to(cum_incl, (RB, RB))
    cum_sub = jnp.dot(D, ONES, preferred_element_type=f32)
    gcnt = jnp.dot(jnp.ones((1, RB), f32), (cum_sub <= lane).astype(f32),
                   preferred_element_type=f32)
    g_row = jnp.minimum(gcnt, float(E - 1))
    total_row = jnp.dot(tiles, ONES, preferred_element_type=f32)
    g_row = jnp.where(jj[:1, :].astype(f32) == float(MT), total_row, g_row)
    g_ref[...] = jnp.broadcast_to(g_row, (E, RB)).astype(jnp.int32)

    def loop2(j, carry):
        ohj = oh_ref[pl.ds(j * RB, RB), :]
        rankj = rank_ref[pl.ds(j * RB, RB), :]
        pj = jnp.sum(ohj * (rankj + jnp.broadcast_to(pad_off, (RB, RB))),
                     axis=1, keepdims=True)
        pos_ref[pl.ds(j * RB, RB), :] = pj.astype(jnp.int32)
        return carry

    lax.fori_loop(0, T // RB, loop2, 0)


def _routing_tc(x_flat):
    return pl.pallas_call(
        _routing_body,
        grid=(1,),
        in_specs=[pl.BlockSpec((T, RB), lambda i: (0, 0))],
        out_specs=[
            pl.BlockSpec((T, 1), lambda i: (0, 0)),
            pl.BlockSpec((E, RB), lambda i: (0, 0)),
        ],
        out_shape=[
            jax.ShapeDtypeStruct((T, 1), jnp.int32),
            jax.ShapeDtypeStruct((E, RB), jnp.int32),
        ],
        scratch_shapes=[
            pltpu.VMEM((T, RB), jnp.float32),
            pltpu.VMEM((T, RB), jnp.float32),
            pltpu.VMEM((T, 1), jnp.float32),
        ],
    )(x_flat)


def _sc_scatter_rows(rows, idx3, n_out):
    B, D = rows.shape
    info = plsc.get_sparse_core_info()
    nc, ns = info.num_cores, info.num_subcores
    nw = nc * ns
    b_per_w = B // nw
    nch = idx3.shape[1]
    ch = idx3.shape[2]
    mesh = plsc.VectorSubcoreMesh(core_axis_name="c", subcore_axis_name="s")

    @functools.partial(
        pl.kernel, mesh=mesh,
        out_type=jax.ShapeDtypeStruct((n_out, D), rows.dtype),
        scratch_types=[
            pltpu.VMEM((ch,), jnp.int32),
            pltpu.VMEM((ch, D), rows.dtype),
            pltpu.SemaphoreType.DMA,
        ],
    )
    def k(rows_hbm, idx_hbm, out_hbm, idx_v, rows_v, sem):
        wid = lax.axis_index("s") * nc + lax.axis_index("c")
        base = wid * b_per_w

        def body(j, carry):
            pltpu.sync_copy(idx_hbm.at[wid, j], idx_v)
            pltpu.sync_copy(rows_hbm.at[pl.ds(base + j * ch, ch)], rows_v)
            pltpu.async_copy(rows_v, out_hbm.at[idx_v], sem).wait()
            return carry

        lax.fori_loop(0, nch, body, 0)

    return k(rows, idx3)


def _sc_gather_rows(table, idx):
    B = idx.shape[0]
    D = table.shape[1]
    info = plsc.get_sparse_core_info()
    nc, ns = info.num_cores, info.num_subcores
    nw = nc * ns
    b_per_w = B // nw
    ch = 64 if b_per_w % 64 == 0 else b_per_w
    nch = b_per_w // ch
    mesh = plsc.VectorSubcoreMesh(core_axis_name="c", subcore_axis_name="s")

    @functools.partial(
        pl.kernel, mesh=mesh,
        out_type=jax.ShapeDtypeStruct((B, D), table.dtype),
        scratch_types=[
            pltpu.VMEM((ch,), jnp.int32),
            pltpu.VMEM((ch, D), table.dtype),
            pltpu.SemaphoreType.DMA,
        ],
    )
    def k(table_hbm, idx_hbm, out_hbm, idx_v, rows_v, sem):
        wid = lax.axis_index("s") * nc + lax.axis_index("c")
        base = wid * b_per_w

        def body(i, carry):
            off = base + i * ch
            pltpu.sync_copy(idx_hbm.at[pl.ds(off, ch)], idx_v)
            pltpu.async_copy(table_hbm.at[idx_v], rows_v, sem).wait()
            pltpu.sync_copy(rows_v, out_hbm.at[pl.ds(off, ch)])
            return carry

        lax.fori_loop(0, nch, body, 0)

    return k(table, idx)


def _mlp_pass_body(acc, add_bias):
    def body(g_ref, x_ref, w1_ref, b1_ref, w2_ref, b2_ref, *rest):
        if acc:
            y0_ref, o_ref = rest
        else:
            (o_ref,) = rest

        @pl.when(pl.program_id(0) < g_ref[MT])
        def _():
            xb = x_ref[...].astype(jnp.bfloat16)
            y = jnp.zeros((TM, H), jnp.float32)
            if add_bias:
                y = y + (0.5 * b2_ref[0, 0])[None, :]
            if acc:
                y = y + y0_ref[...]
            h = jnp.dot(xb, w1_ref[0], preferred_element_type=jnp.float32)
            h = (jax.nn.relu(h + b1_ref[0, 0][None, :]) * 0.5).astype(jnp.bfloat16)
            y = y + jnp.dot(h, w2_ref[0], preferred_element_type=jnp.float32)
            o_ref[...] = y
    return body


def _mlp_pass(g_tile, x_s, W1, b1r, W2, b2r, shift, y0=None):
    grid = (MT,)

    def e_idx(m, g):
        return lax.rem(g[m] + shift, E)

    in_specs = [
        pl.BlockSpec((TM, H), lambda m, g: (m, 0)),
        pl.BlockSpec((1, H, FF), lambda m, g: (e_idx(m, g), 0, 0)),
        pl.BlockSpec((1, 1, FF), lambda m, g: (e_idx(m, g), 0, 0)),
        pl.BlockSpec((1, FF, H), lambda m, g: (e_idx(m, g), 0, 0)),
        pl.BlockSpec((1, 1, H), lambda m, g: (e_idx(m, g), 0, 0)),
    ]
    args = [g_tile, x_s, W1, b1r, W2, b2r]
    if y0 is not None:
        in_specs.append(pl.BlockSpec((TM, H), lambda m, g: (m, 0)))
        args.append(y0)
    grid_spec = pltpu.PrefetchScalarGridSpec(
        num_scalar_prefetch=1,
        grid=grid,
        in_specs=in_specs,
        out_specs=pl.BlockSpec((TM, H), lambda m, g: (m, 0)),
    )
    return pl.pallas_call(
        _mlp_pass_body(y0 is not None, True),
        grid_spec=grid_spec,
        out_shape=jax.ShapeDtypeStruct((APAD, H), jnp.float32),
        compiler_params=pltpu.CompilerParams(
            dimension_semantics=("arbitrary",)),
    )(*args)


def kernel(x, W1, b1, W2, b2):
    B, S, _ = x.shape
    x_flat = x.reshape(-1, H)

    pos_col, g_out = _routing_tc(x_flat)
    pos = pos_col.reshape(T)
    g_tile = g_out[0, :MT + E]

    nw = 32
    idx3 = pos.reshape(nw, 2, T // nw // 2)
    x_s = _sc_scatter_rows(x_flat, idx3, APAD)

    w1b = W1.astype(jnp.bfloat16)
    b1r = b1.reshape(E, 1, FF)
    b2r = b2.reshape(E, 1, H)
    y0 = _mlp_pass(g_tile, x_s, w1b, b1r, W2, b2r, 0)
    y_s = _mlp_pass(g_tile, x_s, w1b, b1r, W2, b2r, 1, y0)

    out_flat = _sc_gather_rows(y_s, pos)
    return out_flat.reshape(B, S, H)
